# baseline (speedup 1.0000x reference)
import functools

import jax
import jax.numpy as jnp
from jax import lax
from jax.experimental import pallas as pl
from jax.experimental.pallas import tpu as pltpu
from jax.experimental.pallas import tpu_sc as plsc

N = 10000
E = 320000
F1 = 128
F2 = 64
NC, NS, L = 2, 16, 16
NW = NC * NS
CHUNK = 64
CPT = 160
EPT = CHUNK * CPT
EPAD = EPT * NW
NPAD = 10240
ROWS_PT = NPAD // NS

_mesh = plsc.VectorSubcoreMesh(core_axis_name="c", subcore_axis_name="s")


def _deg_body(dst_hbm, deg_out, dst_v, ones_v, zeros_v, acc):
    cid = lax.axis_index("c")
    sid = lax.axis_index("s")
    wid = sid * NC + cid
    pltpu.sync_copy(dst_hbm.at[wid], dst_v)
    zeros16 = jnp.zeros((L,), jnp.float32)
    ones16 = jnp.ones((L,), jnp.float32)

    def ob(i, carry):
        ones_v[pl.ds(i * L, L)] = ones16
        return carry

    lax.fori_loop(0, CHUNK // L, ob, 0)

    def zb(i, carry):
        zeros_v[pl.ds(i * L, L)] = zeros16
        return carry

    lax.fori_loop(0, ROWS_PT // L, zb, 0)

    base = sid * ROWS_PT
    pltpu.sync_copy(zeros_v, acc.at[pl.ds(base, ROWS_PT)])
    plsc.subcore_barrier()

    def hb(j, carry):
        pltpu.sync_copy(ones_v, acc.at[dst_v.at[j]], add=True)
        return carry

    lax.fori_loop(0, CPT, hb, 0)
    plsc.subcore_barrier()
    pltpu.sync_copy(acc.at[pl.ds(base, ROWS_PT)],
                    deg_out.at[cid, 0, pl.ds(base, ROWS_PT)])


_deg_call = pl.kernel(
    _deg_body,
    out_type=jax.ShapeDtypeStruct((NC, 1, NPAD), jnp.float32),
    mesh=_mesh,
    scratch_types=[
        pltpu.VMEM((CPT, CHUNK), jnp.int32),
        pltpu.VMEM((CHUNK,), jnp.float32),
        pltpu.VMEM((ROWS_PT,), jnp.float32),
        pltpu.VMEM_SHARED((NPAD,), jnp.float32),
    ],
)


NBUF = 4
SEG = 16
SEGLEN = SEG * CHUNK
CPT0 = 192
CPT1 = 128
EPT0 = CPT0 * CHUNK
EPT1 = CPT1 * CHUNK
CPTMAX = max(CPT0, CPT1)


def _make_agg(F):
    def body(y_hbm, src_hbm, dst_hbm, out_hbm, sidx_v, didx_v, buf, acc,
             gs0, gs1, gs2, gs3, is0, is1):
        gsems = (gs0, gs1, gs2, gs3)
        isems = (is0, is1)
        cid = lax.axis_index("c")
        sid = lax.axis_index("s")
        cpt = jnp.where(cid == 0, CPT0, CPT1)
        nseg = cpt // SEG

        zeros16 = jnp.zeros((L,), jnp.float32)

        def zb(i, carry):
            buf[0, i // (F // L), pl.ds((i % (F // L)) * L, L)] = zeros16
            return carry

        lax.fori_loop(0, CHUNK * (F // L), zb, 0)

        base = sid * ROWS_PT
        for k in range(ROWS_PT // CHUNK):
            pltpu.sync_copy(buf.at[0], acc.at[pl.ds(base + k * CHUNK, CHUNK)])
        plsc.subcore_barrier()

        def load_idx(g, p):
            pltpu.async_copy(src_hbm.at[cid, sid, pl.ds(g * SEGLEN, SEGLEN)],
                             sidx_v.at[pl.ds(p * SEGLEN, SEGLEN)], isems[p])
            pltpu.async_copy(dst_hbm.at[cid, sid, pl.ds(g * SEG, SEG)],
                             didx_v.at[pl.ds(p * SEG, SEG)], isems[p])

        def wait_idx(p):
            pltpu.make_async_copy(
                src_hbm.at[cid, sid, pl.ds(0, SEGLEN)],
                sidx_v.at[pl.ds(p * SEGLEN, SEGLEN)], isems[p]).wait()
            pltpu.make_async_copy(
                dst_hbm.at[cid, sid, pl.ds(0, SEG)],
                didx_v.at[pl.ds(p * SEG, SEG)], isems[p]).wait()

        def gather(p, jl, b):
            idx = sidx_v.at[pl.ds(p * SEGLEN + jl * CHUNK, CHUNK)]
            return pltpu.async_copy(y_hbm.at[idx], buf.at[b], gsems[b])

        def wait_gather(p, jl, b):
            pltpu.make_async_copy(
                y_hbm.at[sidx_v.at[pl.ds(p * SEGLEN + jl * CHUNK, CHUNK)]],
                buf.at[b], gsems[b]).wait()

        load_idx(0, 0)

        def pair_body(g2, carry):
            for p in range(2):
                g = g2 * 2 + p
                wait_idx(p)

                for b in range(NBUF):
                    gather(p, b, b)

                @pl.when(g + 1 < nseg)
                def _():
                    load_idx(g + 1, 1 - p)

                def grp(gg, carry2):
                    for b in range(NBUF):
                        jl = gg * NBUF + b
                        wait_gather(p, jl, b)
                        pltpu.sync_copy(buf.at[b],
                                        acc.at[didx_v.at[p * SEG + jl]],
                                        add=True)

                        @pl.when(jl + NBUF < SEG)
                        def _():
                            gather(p, jl + NBUF, b)

                    return carry2

                lax.fori_loop(0, SEG // NBUF, grp, 0)
            return carry

        lax.fori_loop(0, nseg // 2, pair_body, 0)
        plsc.subcore_barrier()
        pltpu.sync_copy(acc.at[pl.ds(base, ROWS_PT)],
                        out_hbm.at[cid, pl.ds(base, ROWS_PT)])

    return pl.kernel(
        body,
        out_type=jax.ShapeDtypeStruct((NC, NPAD, F), jnp.float32),
        mesh=_mesh,
        scratch_types=[
            pltpu.VMEM((2 * SEGLEN,), jnp.int32),
            pltpu.VMEM((2 * SEG, CHUNK), jnp.int32),
            pltpu.VMEM((NBUF, CHUNK, F), jnp.float32),
            pltpu.VMEM_SHARED((NPAD, F), jnp.float32),
        ] + [pltpu.SemaphoreType.DMA] * (NBUF + 2),
    )


_agg128 = _make_agg(F1)


R = 1000


def _tc_a_body(deg_ref, x_ref, w_ref, y_ref, dis_ref):
    degs = jnp.sum(deg_ref[...], axis=1, keepdims=True) + 1.0
    dis = lax.rsqrt(degs)
    xw = jnp.dot(x_ref[...], w_ref[...], preferred_element_type=jnp.float32)
    y_ref[...] = xw * dis
    dis_ref[...] = dis


_tc_a = pl.pallas_call(
    _tc_a_body,
    grid=(N // R,),
    in_specs=[
        pl.BlockSpec((R, NC), lambda i: (i, 0)),
        pl.BlockSpec((R, F1), lambda i: (i, 0)),
        pl.BlockSpec((F1, F1), lambda i: (0, 0)),
    ],
    out_specs=[
        pl.BlockSpec((R, F1), lambda i: (i, 0)),
        pl.BlockSpec((R, 1), lambda i: (i, 0)),
    ],
    out_shape=[
        jax.ShapeDtypeStruct((N, F1), jnp.float32),
        jax.ShapeDtypeStruct((N, 1), jnp.float32),
    ],
)


def _tc_b_body(p_ref, y1_ref, dis_ref, b1_ref, w2_ref, y2_ref):
    s = p_ref[0] + p_ref[1] + y1_ref[...]
    h = jnp.maximum(s * dis_ref[...] + b1_ref[...], 0.0)
    y2_ref[...] = jnp.dot(h, w2_ref[...],
                          preferred_element_type=jnp.float32) * dis_ref[...]


_tc_b = pl.pallas_call(
    _tc_b_body,
    grid=(N // R,),
    in_specs=[
        pl.BlockSpec((2, R, F1), lambda i: (0, i, 0)),
        pl.BlockSpec((R, F1), lambda i: (i, 0)),
        pl.BlockSpec((R, 1), lambda i: (i, 0)),
        pl.BlockSpec((1, F1), lambda i: (0, 0)),
        pl.BlockSpec((F1, F1), lambda i: (0, 0)),
    ],
    out_specs=pl.BlockSpec((R, F1), lambda i: (i, 0)),
    out_shape=jax.ShapeDtypeStruct((N, F1), jnp.float32),
)


def _tc_c_body(q_ref, y2_ref, dis_ref, b2_ref, o_ref):
    t = (q_ref[0] + q_ref[1] + y2_ref[...]) * dis_ref[...]
    z = t[:, :F2] + b2_ref[...]
    m = jnp.max(z, axis=1, keepdims=True)
    e = jnp.exp(z - m)
    ssum = jnp.sum(e, axis=1, keepdims=True)
    o_ref[...] = z - m - jnp.log(ssum)


_tc_c = pl.pallas_call(
    _tc_c_body,
    grid=(N // R,),
    in_specs=[
        pl.BlockSpec((2, R, F1), lambda i: (0, i, 0)),
        pl.BlockSpec((R, F1), lambda i: (i, 0)),
        pl.BlockSpec((R, 1), lambda i: (i, 0)),
        pl.BlockSpec((1, F2), lambda i: (0, 0)),
    ],
    out_specs=pl.BlockSpec((R, F2), lambda i: (i, 0)),
    out_shape=jax.ShapeDtypeStruct((N, F2), jnp.float32),
)


@jax.jit
def kernel(x, edge_index, W1, b1, W2, b2):
    ei = edge_index.astype(jnp.int32)
    src = ei[0]
    dst = ei[1]
    padlen = EPAD - E
    src_flat = jnp.concatenate([src, jnp.zeros((padlen,), jnp.int32)])
    dst_flat = jnp.concatenate([dst, jnp.full((padlen,), N, jnp.int32)])
    dstp_deg = dst_flat.reshape(NW, CPT, CHUNK)
    cut = NS * EPT0
    def rows(flat, ept, fill):
        r = flat.reshape(NS, ept)
        if ept < CPTMAX * CHUNK:
            r = jnp.concatenate(
                [r, jnp.full((NS, CPTMAX * CHUNK - ept), fill, jnp.int32)], axis=1)
        return r
    src_all = jnp.stack([rows(src_flat[:cut], EPT0, 0),
                         rows(src_flat[cut:], EPT1, 0)])
    dst_all = src_all
    dst_all = jnp.stack([rows(dst_flat[:cut], EPT0, N),
                         rows(dst_flat[cut:], EPT1, N)]).reshape(
                             NC, NS, CPTMAX, CHUNK)

    degp = _deg_call(dstp_deg)
    degT = jnp.transpose(degp.reshape(NC, NPAD))
    y1, dis = _tc_a(degT, x, W1)
    P = _agg128(y1, src_all, dst_all)
    w2p = jnp.pad(W2, ((0, 0), (0, F1 - F2)))
    y2 = _tc_b(P, y1, dis, b1.reshape(1, F1), w2p)
    Q = _agg128(y2, src_all, dst_all)
    return _tc_c(Q, y2, dis, b2.reshape(1, F2))


# speedup vs baseline: 1.1568x; 1.1568x over previous
import functools

import jax
import jax.numpy as jnp
from jax import lax
from jax.experimental import pallas as pl
from jax.experimental.pallas import tpu as pltpu
from jax.experimental.pallas import tpu_sc as plsc

N = 10000
E = 320000
F1 = 128
F2 = 64
NC, NS, L = 2, 16, 16
NW = NC * NS
CHUNK = 64
CPT = 160
EPT = CHUNK * CPT
EPAD = EPT * NW
NPAD = 10240
ROWS_PT = NPAD // NS

_mesh = plsc.VectorSubcoreMesh(core_axis_name="c", subcore_axis_name="s")


def _deg_body(dst_hbm, deg_out, dst_v, ones_v, zeros_v, acc):
    cid = lax.axis_index("c")
    sid = lax.axis_index("s")
    wid = sid * NC + cid
    pltpu.sync_copy(dst_hbm.at[wid], dst_v)
    zeros16 = jnp.zeros((L,), jnp.float32)
    ones16 = jnp.ones((L,), jnp.float32)

    def ob(i, carry):
        ones_v[pl.ds(i * L, L)] = ones16
        return carry

    lax.fori_loop(0, CHUNK // L, ob, 0)

    def zb(i, carry):
        zeros_v[pl.ds(i * L, L)] = zeros16
        return carry

    lax.fori_loop(0, ROWS_PT // L, zb, 0)

    base = sid * ROWS_PT
    pltpu.sync_copy(zeros_v, acc.at[pl.ds(base, ROWS_PT)])
    plsc.subcore_barrier()

    def hb(j, carry):
        pltpu.sync_copy(ones_v, acc.at[dst_v.at[j]], add=True)
        return carry

    lax.fori_loop(0, CPT, hb, 0)
    plsc.subcore_barrier()
    pltpu.sync_copy(acc.at[pl.ds(base, ROWS_PT)],
                    deg_out.at[cid, 0, pl.ds(base, ROWS_PT)])


_deg_call = pl.kernel(
    _deg_body,
    out_type=jax.ShapeDtypeStruct((NC, 1, NPAD), jnp.float32),
    mesh=_mesh,
    scratch_types=[
        pltpu.VMEM((CPT, CHUNK), jnp.int32),
        pltpu.VMEM((CHUNK,), jnp.float32),
        pltpu.VMEM((ROWS_PT,), jnp.float32),
        pltpu.VMEM_SHARED((NPAD,), jnp.float32),
    ],
)


NBUF = 4
SEG = 16
SEGLEN = SEG * CHUNK
CPT0 = 208
CPT1 = 112
EPT0 = CPT0 * CHUNK
EPT1 = CPT1 * CHUNK
CPTMAX = max(CPT0, CPT1)


def _make_agg(F):
    def body(y_hbm, src_hbm, dst_hbm, out_hbm, sidx_v, didx_v, tidx_v, buf,
             acc, gs0, gs1, gs2, gs3, ss0, ss1, ss2, ss3, is0, is1):
        gsems = (gs0, gs1, gs2, gs3)
        ssems = (ss0, ss1, ss2, ss3)
        isems = (is0, is1)
        cid = lax.axis_index("c")
        sid = lax.axis_index("s")
        cpt = jnp.where(cid == 0, CPT0, CPT1)
        nseg = cpt // SEG

        zeros16 = jnp.zeros((L,), jnp.float32)

        def zb(i, carry):
            buf[0, i // (F // L), pl.ds((i % (F // L)) * L, L)] = zeros16
            return carry

        lax.fori_loop(0, CHUNK * (F // L), zb, 0)

        base = sid * ROWS_PT
        for k in range(ROWS_PT // CHUNK):
            pltpu.sync_copy(buf.at[0], acc.at[pl.ds(base + k * CHUNK, CHUNK)])
        plsc.subcore_barrier()

        def load_idx(g, p):
            pltpu.async_copy(src_hbm.at[cid, sid, pl.ds(g * SEGLEN, SEGLEN)],
                             sidx_v.at[pl.ds(p * SEGLEN, SEGLEN)], isems[p])
            pltpu.async_copy(dst_hbm.at[cid, sid, pl.ds(g * SEG, SEG)],
                             didx_v.at[pl.ds(p * SEG, SEG)], isems[p])

        def wait_idx(p):
            pltpu.make_async_copy(
                src_hbm.at[cid, sid, pl.ds(0, SEGLEN)],
                sidx_v.at[pl.ds(p * SEGLEN, SEGLEN)], isems[p]).wait()
            pltpu.make_async_copy(
                dst_hbm.at[cid, sid, pl.ds(0, SEG)],
                didx_v.at[pl.ds(p * SEG, SEG)], isems[p]).wait()

        def gather(p, jl, b):
            idx = sidx_v.at[pl.ds(p * SEGLEN + jl * CHUNK, CHUNK)]
            return pltpu.async_copy(y_hbm.at[idx], buf.at[b], gsems[b])

        def wait_gather(p, jl, b):
            pltpu.make_async_copy(
                y_hbm.at[sidx_v.at[pl.ds(p * SEGLEN + jl * CHUNK, CHUNK)]],
                buf.at[b], gsems[b]).wait()

        def wait_scatter(b):
            pltpu.make_async_copy(buf.at[b], acc.at[tidx_v],
                                  ssems[b]).wait()

        nn16 = jnp.full((L,), N, jnp.int32)

        def tb(i, carry):
            tidx_v[pl.ds(i * L, L)] = nn16
            return carry

        lax.fori_loop(0, CHUNK // L, tb, 0)
        for b in range(NBUF):
            pltpu.async_copy(buf.at[0], acc.at[tidx_v], ssems[b], add=True)

        load_idx(0, 0)

        def pair_body(g2, carry):
            for p in range(2):
                g = g2 * 2 + p
                wait_idx(p)

                for b in range(2):
                    wait_scatter(b)
                    gather(p, b, b)

                @pl.when(g + 1 < nseg)
                def _():
                    load_idx(g + 1, 1 - p)

                def grp(gg, carry2):
                    for b in range(NBUF):
                        jl = gg * NBUF + b
                        wait_gather(p, jl, b)
                        pltpu.async_copy(buf.at[b],
                                         acc.at[didx_v.at[p * SEG + jl]],
                                         ssems[b], add=True)
                        if True:
                            b2 = (b + 2) % NBUF

                            @pl.when(jl + 2 < SEG)
                            def _():
                                wait_scatter(b2)
                                gather(p, jl + 2, b2)

                    return carry2

                lax.fori_loop(0, SEG // NBUF, grp, 0)
            return carry

        lax.fori_loop(0, nseg // 2, pair_body, 0)
        for b in range(NBUF):
            wait_scatter(b)
        plsc.subcore_barrier()
        pltpu.sync_copy(acc.at[pl.ds(base, ROWS_PT)],
                        out_hbm.at[cid, pl.ds(base, ROWS_PT)])

    return pl.kernel(
        body,
        out_type=jax.ShapeDtypeStruct((NC, NPAD, F), jnp.float32),
        mesh=_mesh,
        scratch_types=[
            pltpu.VMEM((2 * SEGLEN,), jnp.int32),
            pltpu.VMEM((2 * SEG, CHUNK), jnp.int32),
            pltpu.VMEM((CHUNK,), jnp.int32),
            pltpu.VMEM((NBUF, CHUNK, F), jnp.float32),
            pltpu.VMEM_SHARED((NPAD, F), jnp.float32),
        ] + [pltpu.SemaphoreType.DMA] * (2 * NBUF + 2),
    )


_agg128 = _make_agg(F1)


R = 1000


def _tc_a_body(deg_ref, x_ref, w_ref, y_ref, dis_ref):
    degs = jnp.sum(deg_ref[...], axis=1, keepdims=True) + 1.0
    dis = lax.rsqrt(degs)
    xw = jnp.dot(x_ref[...], w_ref[...], preferred_element_type=jnp.float32)
    y_ref[...] = xw * dis
    dis_ref[...] = dis


_tc_a = pl.pallas_call(
    _tc_a_body,
    grid=(N // R,),
    in_specs=[
        pl.BlockSpec((R, NC), lambda i: (i, 0)),
        pl.BlockSpec((R, F1), lambda i: (i, 0)),
        pl.BlockSpec((F1, F1), lambda i: (0, 0)),
    ],
    out_specs=[
        pl.BlockSpec((R, F1), lambda i: (i, 0)),
        pl.BlockSpec((R, 1), lambda i: (i, 0)),
    ],
    out_shape=[
        jax.ShapeDtypeStruct((N, F1), jnp.float32),
        jax.ShapeDtypeStruct((N, 1), jnp.float32),
    ],
)


def _tc_b_body(p_ref, y1_ref, dis_ref, b1_ref, w2_ref, y2_ref):
    s = p_ref[0] + p_ref[1] + y1_ref[...]
    h = jnp.maximum(s * dis_ref[...] + b1_ref[...], 0.0)
    y2_ref[...] = jnp.dot(h, w2_ref[...],
                          preferred_element_type=jnp.float32) * dis_ref[...]


_tc_b = pl.pallas_call(
    _tc_b_body,
    grid=(N // R,),
    in_specs=[
        pl.BlockSpec((2, R, F1), lambda i: (0, i, 0)),
        pl.BlockSpec((R, F1), lambda i: (i, 0)),
        pl.BlockSpec((R, 1), lambda i: (i, 0)),
        pl.BlockSpec((1, F1), lambda i: (0, 0)),
        pl.BlockSpec((F1, F1), lambda i: (0, 0)),
    ],
    out_specs=pl.BlockSpec((R, F1), lambda i: (i, 0)),
    out_shape=jax.ShapeDtypeStruct((N, F1), jnp.float32),
)


def _tc_c_body(q_ref, y2_ref, dis_ref, b2_ref, o_ref):
    t = (q_ref[0] + q_ref[1] + y2_ref[...]) * dis_ref[...]
    z = t[:, :F2] + b2_ref[...]
    m = jnp.max(z, axis=1, keepdims=True)
    e = jnp.exp(z - m)
    ssum = jnp.sum(e, axis=1, keepdims=True)
    o_ref[...] = z - m - jnp.log(ssum)


_tc_c = pl.pallas_call(
    _tc_c_body,
    grid=(N // R,),
    in_specs=[
        pl.BlockSpec((2, R, F1), lambda i: (0, i, 0)),
        pl.BlockSpec((R, F1), lambda i: (i, 0)),
        pl.BlockSpec((R, 1), lambda i: (i, 0)),
        pl.BlockSpec((1, F2), lambda i: (0, 0)),
    ],
    out_specs=pl.BlockSpec((R, F2), lambda i: (i, 0)),
    out_shape=jax.ShapeDtypeStruct((N, F2), jnp.float32),
)


@jax.jit
def kernel(x, edge_index, W1, b1, W2, b2):
    ei = edge_index.astype(jnp.int32)
    src = ei[0]
    dst = ei[1]
    padlen = EPAD - E
    src_flat = jnp.concatenate([src, jnp.zeros((padlen,), jnp.int32)])
    dst_flat = jnp.concatenate([dst, jnp.full((padlen,), N, jnp.int32)])
    dstp_deg = dst_flat.reshape(NW, CPT, CHUNK)
    cut = NS * EPT0
    def rows(flat, ept, fill):
        r = flat.reshape(NS, ept)
        if ept < CPTMAX * CHUNK:
            r = jnp.concatenate(
                [r, jnp.full((NS, CPTMAX * CHUNK - ept), fill, jnp.int32)], axis=1)
        return r
    src_all = jnp.stack([rows(src_flat[:cut], EPT0, 0),
                         rows(src_flat[cut:], EPT1, 0)])
    dst_all = src_all
    dst_all = jnp.stack([rows(dst_flat[:cut], EPT0, N),
                         rows(dst_flat[cut:], EPT1, N)]).reshape(
                             NC, NS, CPTMAX, CHUNK)

    degp = _deg_call(dstp_deg)
    degT = jnp.transpose(degp.reshape(NC, NPAD))
    y1, dis = _tc_a(degT, x, W1)
    P = _agg128(y1, src_all, dst_all)
    w2p = jnp.pad(W2, ((0, 0), (0, F1 - F2)))
    y2 = _tc_b(P, y1, dis, b1.reshape(1, F1), w2p)
    Q = _agg128(y2, src_all, dst_all)
    return _tc_c(Q, y2, dis, b2.reshape(1, F2))
